# baseline (speedup 1.0000x reference)
import jax
import jax.numpy as jnp
from jax.experimental import pallas as pl
from jax.experimental.pallas import tpu as pltpu

RHO_W = 1000.0
RHO_I = 917.0
G = 9.81
SEC_PER_A = 31556926.0
DX = 100.0
CELL_AREA = DX * DX
SHEET_K = 0.01
SHEET_EXP = 1.25
STEP_H = 0.1
SPACING = 2.0
CLOSURE = 5e-25
NEXP = 3


def _residual_body(pot_h, pot_t, th_h, th_t, dn_h, dn_t, gate, out):
    h_link = 0.5 * (th_h[...] + th_t[...])
    gradient = (pot_h[...] - pot_t[...]) / DX
    flux = -SHEET_K * h_link ** SHEET_EXP * (jnp.abs(gradient) + 1e-12) ** (-0.5) * gradient
    flux = jnp.where(gate[...] != 0, 0.0, flux)
    discharge = 0.5 * (dn_h[...] + dn_t[...])
    out[...] = jnp.abs(flux - discharge)


def kernel(edge_index, adjacent_nodes, status_at_node, bedrock_elevation,
           overburden_pressure, melt_rate, surface_melt_rate, sliding_velocity):
    head = edge_index[0]
    tail = edge_index[1]
    N = bedrock_elevation.shape[0]
    E = head.shape[0]

    specific_melt = melt_rate * (RHO_W / RHO_I) / SEC_PER_A + surface_melt_rate
    base_pot = RHO_W * G * bedrock_elevation + overburden_pressure

    valid = adjacent_nodes != -1
    adj_pot = jnp.mean(jnp.where(valid, base_pot[adjacent_nodes], 0.0), axis=1)
    inflow_outflow = jnp.where(base_pot > adj_pot,
                               1 * (status_at_node > 0),
                               -1 * (status_at_node > 0))

    local = specific_melt * CELL_AREA
    bp_h = base_pot[head]
    bp_t = base_pot[tail]
    downhill_to_head = bp_t > bp_h
    recv = jnp.where(downhill_to_head, head, tail)
    send = jnp.where(downhill_to_head, tail, head)
    discharge_node = local + jnp.zeros(N, dtype=local.dtype).at[recv].add(local[send])
    discharge_node = jnp.where(status_at_node != 0, 0.0, discharge_node)

    vsum = jnp.zeros(N, dtype=sliding_velocity.dtype).at[head].add(sliding_velocity).at[tail].add(sliding_velocity)
    vcnt = jnp.zeros(N, dtype=sliding_velocity.dtype).at[head].add(1.0).at[tail].add(1.0)
    sliding_node = jnp.abs(vsum / jnp.maximum(vcnt, 1.0)) / SEC_PER_A

    potential = base_pot - overburden_pressure
    pressure = overburden_pressure
    thickness = sliding_node ** 2 * STEP_H / (CLOSURE * pressure ** NEXP * SPACING ** 2 + 1e-30)

    gate = ((inflow_outflow[head] == 1) | (inflow_outflow[tail] == 1)).astype(jnp.int32)

    B = 512
    grid = (E // B,)
    spec = pl.BlockSpec((B,), lambda i: (i,))
    out = pl.pallas_call(
        _residual_body,
        grid=grid,
        in_specs=[spec] * 7,
        out_specs=spec,
        out_shape=jax.ShapeDtypeStruct((E,), jnp.float32),
    )(potential[head], potential[tail], thickness[head], thickness[tail],
      discharge_node[head], discharge_node[tail], gate)
    return out


# speedup vs baseline: 241.9853x; 241.9853x over previous
import functools

import jax
import jax.numpy as jnp
from jax import lax
from jax.experimental import pallas as pl
from jax.experimental.pallas import tpu as pltpu
from jax.experimental.pallas import tpu_sc as plsc

RHO_W = 1000.0
RHO_I = 917.0
G = 9.81
SEC_PER_A = 31556926.0
DX = 100.0
CELL_AREA = DX * DX
SHEET_K = 0.01
STEP_H = 0.1
SPACING = 2.0
CLOSURE = 5e-25
NEXP = 3

N = 100000
E = 1600000
K_ADJ = 8

NC = 2
NS = 16
NW = NC * NS
NPAD = 100352
NSL = NPAD // NW
EW = E // NW
C = 2000
NCH = EW // C
L = 16


def _mesh():
    return plsc.VectorSubcoreMesh(core_axis_name="c", subcore_axis_name="s",
                                  num_cores=NC, num_subcores=NS)


def _wid():
    return lax.axis_index("s") * NC + lax.axis_index("c")


def _rsqrt(x):
    i = plsc.bitcast(x, jnp.int32)
    i = 0x5F3759DF - lax.shift_right_logical(i, 1)
    y = plsc.bitcast(i, jnp.float32)
    for _ in range(3):
        y = y * (1.5 - 0.5 * x * y * y)
    return y


def _k1_body(bed, ob, melt, smelt, bp_out, loc_out, pot_out,
             bedv, obv, meltv, smeltv, bpv, locv, potv):
    w = _wid()
    off = w * NSL
    pltpu.sync_copy(bed.at[pl.ds(off, NSL)], bedv)
    pltpu.sync_copy(ob.at[pl.ds(off, NSL)], obv)
    pltpu.sync_copy(melt.at[pl.ds(off, NSL)], meltv)
    pltpu.sync_copy(smelt.at[pl.ds(off, NSL)], smeltv)

    def body(i, _):
        s = pl.ds(i * L, L)
        b = bedv[s]
        o = obv[s]
        bp = RHO_W * G * b + o
        bpv[s] = bp
        potv[s] = bp - o
        locv[s] = (meltv[s] * (RHO_W / RHO_I / SEC_PER_A) + smeltv[s]) * CELL_AREA
        return 0

    lax.fori_loop(0, NSL // L, body, 0)
    pltpu.sync_copy(bpv, bp_out.at[pl.ds(off, NSL)])
    pltpu.sync_copy(locv, loc_out.at[pl.ds(off, NSL)])
    pltpu.sync_copy(potv, pot_out.at[pl.ds(off, NSL)])


def _k2_body(bp, adjt, status, if_out, bptab, adjv, statv, accv, outv):
    w = _wid()
    off = w * NSL
    pltpu.sync_copy(bp, bptab)
    pltpu.sync_copy(status.at[pl.ds(off, NSL)], statv)

    def zero(i, _):
        accv[pl.ds(i * L, L)] = jnp.zeros((L,), jnp.float32)
        return 0

    lax.fori_loop(0, NSL // L, zero, 0)

    def per_j(j, _):
        joff = pl.multiple_of(j * NPAD + off, 8)
        pltpu.sync_copy(adjt.at[pl.ds(joff, NSL)], adjv)

        def per_i(i, _):
            s = pl.ds(i * L, L)
            idx = adjv[s]
            accv[s] = accv[s] + plsc.load_gather(bptab, [idx])
            return 0

        lax.fori_loop(0, NSL // L, per_i, 0)
        return 0

    lax.fori_loop(0, K_ADJ, per_j, 0)

    def fin(i, _):
        s = pl.ds(i * L, L)
        adj_pot = accv[s] * (1.0 / K_ADJ)
        mybp = bptab[pl.ds(off + i * L, L)]
        sign = jnp.where(mybp > adj_pot, 1.0, -1.0)
        outv[s] = jnp.where(statv[s] > 0, sign, 0.0)
        return 0

    lax.fori_loop(0, NSL // L, fin, 0)
    pltpu.sync_copy(outv, if_out.at[pl.ds(off, NSL)])


def _k3_body(head, tail, slide, bp, loc,
             dis_out, vs_out, vc_out,
             headv, tailv, slidev, recvv, sendv, lsendv, onesv, zv,
             bphv, bptv, bouncev, bp_sh, loc_sh, dis_sh, vs_sh, vc_sh):
    c = lax.axis_index("c")
    s = lax.axis_index("s")
    w = s * NC + c
    noff = s * NSL

    def zfill(i, _):
        zv[pl.ds(i * L, L)] = jnp.zeros((L,), jnp.float32)
        onesv[pl.ds(i * L, L)] = jnp.full((L,), 1.0, jnp.float32)
        return 0

    lax.fori_loop(0, C // L, zfill, 0)

    def stripe2(k, _):
        soff = noff + k * 1568
        pltpu.sync_copy(zv.at[pl.ds(0, 1568)], dis_sh.at[pl.ds(soff, 1568)])
        pltpu.sync_copy(zv.at[pl.ds(0, 1568)], vs_sh.at[pl.ds(soff, 1568)])
        pltpu.sync_copy(zv.at[pl.ds(0, 1568)], vc_sh.at[pl.ds(soff, 1568)])
        pltpu.sync_copy(loc.at[pl.ds(soff, 1568)], bouncev)
        pltpu.sync_copy(bouncev, loc_sh.at[pl.ds(soff, 1568)])
        pltpu.sync_copy(bp.at[pl.ds(soff, 1568)], bouncev)
        pltpu.sync_copy(bouncev, bp_sh.at[pl.ds(soff, 1568)])
        return 0

    lax.fori_loop(0, 2, stripe2, 0)
    plsc.subcore_barrier()

    def chunk(ci, _):
        off = w * EW + ci * C
        pltpu.sync_copy(head.at[pl.ds(off, C)], headv)
        pltpu.sync_copy(tail.at[pl.ds(off, C)], tailv)
        pltpu.sync_copy(slide.at[pl.ds(off, C)], slidev)

        pltpu.sync_copy(bp_sh.at[headv], bphv)
        pltpu.sync_copy(bp_sh.at[tailv], bptv)

        def vb(i, _):
            sl = pl.ds(i * L, L)
            h = headv[sl]
            t = tailv[sl]
            down = bptv[sl] > bphv[sl]
            recvv[sl] = jnp.where(down, h, t)
            sendv[sl] = jnp.where(down, t, h)
            return 0

        lax.fori_loop(0, C // L, vb, 0)

        pltpu.sync_copy(loc_sh.at[sendv], lsendv)
        pltpu.sync_copy(lsendv, dis_sh.at[recvv], add=True)
        pltpu.sync_copy(slidev, vs_sh.at[headv], add=True)
        pltpu.sync_copy(slidev, vs_sh.at[tailv], add=True)
        pltpu.sync_copy(onesv, vc_sh.at[headv], add=True)
        pltpu.sync_copy(onesv, vc_sh.at[tailv], add=True)
        return 0

    lax.fori_loop(0, NCH, chunk, 0)
    plsc.subcore_barrier()

    def out2(k, _):
        soff = noff + k * 1568
        hoff = c * NPAD + soff
        pltpu.sync_copy(dis_sh.at[pl.ds(soff, 1568)], bouncev)
        pltpu.sync_copy(bouncev, dis_out.at[pl.ds(hoff, 1568)])
        pltpu.sync_copy(vs_sh.at[pl.ds(soff, 1568)], bouncev)
        pltpu.sync_copy(bouncev, vs_out.at[pl.ds(hoff, 1568)])
        pltpu.sync_copy(vc_sh.at[pl.ds(soff, 1568)], bouncev)
        pltpu.sync_copy(bouncev, vc_out.at[pl.ds(hoff, 1568)])
        return 0

    lax.fori_loop(0, 2, out2, 0)


def _k4_body(dis_p, vs_p, vc_p, loc, status, ob, dn_out, th_out,
             d0v, d1v, v0v, v1v, c0v, c1v, locv, statv, obv, dnv, thv):
    w = _wid()
    off = w * NSL
    pltpu.sync_copy(dis_p.at[pl.ds(off, NSL)], d0v)
    pltpu.sync_copy(dis_p.at[pl.ds(NPAD + off, NSL)], d1v)
    pltpu.sync_copy(vs_p.at[pl.ds(off, NSL)], v0v)
    pltpu.sync_copy(vs_p.at[pl.ds(NPAD + off, NSL)], v1v)
    pltpu.sync_copy(vc_p.at[pl.ds(off, NSL)], c0v)
    pltpu.sync_copy(vc_p.at[pl.ds(NPAD + off, NSL)], c1v)
    pltpu.sync_copy(loc.at[pl.ds(off, NSL)], locv)
    pltpu.sync_copy(status.at[pl.ds(off, NSL)], statv)
    pltpu.sync_copy(ob.at[pl.ds(off, NSL)], obv)

    def body(i, _):
        s = pl.ds(i * L, L)
        dn = locv[s] + d0v[s] + d1v[s]
        dnv[s] = jnp.where(statv[s] != 0, 0.0, dn)
        vsum = v0v[s] + v1v[s]
        vcnt = c0v[s] + c1v[s]
        sn = jnp.abs(vsum / jnp.maximum(vcnt, 1.0)) * (1.0 / SEC_PER_A)
        p = obv[s]
        thv[s] = sn * sn * STEP_H / (CLOSURE * p * p * p * (SPACING * SPACING) + 1e-30)
        return 0

    lax.fori_loop(0, NSL // L, body, 0)
    pltpu.sync_copy(dnv, dn_out.at[pl.ds(off, NSL)])
    pltpu.sync_copy(thv, th_out.at[pl.ds(off, NSL)])


def _k5_body(head, tail, pot, th, dn, inf, res_out,
             headv, tailv, phv, ptv, thv, ttv, dhv, dtv, ihv, itv, outv,
             bouncev, pot_sh, th_sh, dn_sh, if_sh):
    c = lax.axis_index("c")
    s = lax.axis_index("s")
    w = s * NC + c
    noff = s * NSL

    def tload(k, _):
        soff = noff + k * 1568
        for hbm_ref, sh_ref in ((pot, pot_sh), (th, th_sh), (dn, dn_sh), (inf, if_sh)):
            pltpu.sync_copy(hbm_ref.at[pl.ds(soff, 1568)], bouncev)
            pltpu.sync_copy(bouncev, sh_ref.at[pl.ds(soff, 1568)])
        return 0

    lax.fori_loop(0, 2, tload, 0)
    plsc.subcore_barrier()

    def chunk(ci, _):
        off = w * EW + ci * C
        pltpu.sync_copy(head.at[pl.ds(off, C)], headv)
        pltpu.sync_copy(tail.at[pl.ds(off, C)], tailv)
        pltpu.sync_copy(pot_sh.at[headv], phv)
        pltpu.sync_copy(pot_sh.at[tailv], ptv)
        pltpu.sync_copy(th_sh.at[headv], thv)
        pltpu.sync_copy(th_sh.at[tailv], ttv)
        pltpu.sync_copy(dn_sh.at[headv], dhv)
        pltpu.sync_copy(dn_sh.at[tailv], dtv)
        pltpu.sync_copy(if_sh.at[headv], ihv)
        pltpu.sync_copy(if_sh.at[tailv], itv)

        def vb(i, _):
            sl = pl.ds(i * L, L)
            hl = 0.5 * (thv[sl] + ttv[sl])
            g = (phv[sl] - ptv[sl]) * (1.0 / DX)
            a = jnp.abs(g) + 1e-12
            r = _rsqrt(a)
            q = _rsqrt(_rsqrt(hl))
            flux = (-SHEET_K) * hl * q * r * g
            gate = (ihv[sl] > 0.5) | (itv[sl] > 0.5)
            flux = jnp.where(gate, 0.0, flux)
            d = 0.5 * (dhv[sl] + dtv[sl])
            outv[sl] = jnp.abs(flux - d)
            return 0

        lax.fori_loop(0, C // L, vb, 0)
        pltpu.sync_copy(outv, res_out.at[pl.ds(off, C)])
        return 0

    lax.fori_loop(0, NCH, chunk, 0)


def _f32(shape):
    return jax.ShapeDtypeStruct(shape, jnp.float32)


def kernel(edge_index, adjacent_nodes, status_at_node, bedrock_elevation,
           overburden_pressure, melt_rate, surface_melt_rate, sliding_velocity):
    head = edge_index[0]
    tail = edge_index[1]
    pad = NPAD - N
    bed_p = jnp.pad(bedrock_elevation, (0, pad))
    ob_p = jnp.pad(overburden_pressure, (0, pad))
    melt_p = jnp.pad(melt_rate, (0, pad))
    smelt_p = jnp.pad(surface_melt_rate, (0, pad))
    stat_p = jnp.pad(status_at_node, (0, pad))
    adjt_p = jnp.pad(adjacent_nodes.T, ((0, 0), (0, pad))).reshape(-1)

    mesh = _mesh()
    cp = pltpu.CompilerParams(needs_layout_passes=False)

    k1 = pl.kernel(
        _k1_body, out_type=(_f32((NPAD,)),) * 3, mesh=mesh, compiler_params=cp,
        scratch_types=[pltpu.VMEM((NSL,), jnp.float32)] * 7,
    )
    base_pot, local, potential = k1(bed_p, ob_p, melt_p, smelt_p)

    k2 = pl.kernel(
        _k2_body, out_type=_f32((NPAD,)), mesh=mesh, compiler_params=cp,
        scratch_types=[
            pltpu.VMEM((NPAD,), jnp.float32),
            pltpu.VMEM((NSL,), jnp.int32),
            pltpu.VMEM((NSL,), jnp.int32),
            pltpu.VMEM((NSL,), jnp.float32),
            pltpu.VMEM((NSL,), jnp.float32),
        ],
    )
    inflow = k2(base_pot, adjt_p, stat_p)

    k3 = pl.kernel(
        _k3_body, out_type=(_f32((NC * NPAD,)),) * 3, mesh=mesh, compiler_params=cp,
        scratch_types=[
            pltpu.VMEM((C,), jnp.int32),
            pltpu.VMEM((C,), jnp.int32),
            pltpu.VMEM((C,), jnp.float32),
            pltpu.VMEM((C,), jnp.int32),
            pltpu.VMEM((C,), jnp.int32),
            pltpu.VMEM((C,), jnp.float32),
            pltpu.VMEM((C,), jnp.float32),
            pltpu.VMEM((C,), jnp.float32),
            pltpu.VMEM((C,), jnp.float32),
            pltpu.VMEM((C,), jnp.float32),
            pltpu.VMEM((1568,), jnp.float32),
            pltpu.VMEM_SHARED((NPAD,), jnp.float32),
            pltpu.VMEM_SHARED((NPAD,), jnp.float32),
            pltpu.VMEM_SHARED((NPAD,), jnp.float32),
            pltpu.VMEM_SHARED((NPAD,), jnp.float32),
            pltpu.VMEM_SHARED((NPAD,), jnp.float32),
        ],
    )
    dis_p, vs_p, vc_p = k3(head, tail, sliding_velocity, base_pot, local)

    k4 = pl.kernel(
        _k4_body, out_type=(_f32((NPAD,)),) * 2, mesh=mesh, compiler_params=cp,
        scratch_types=(
            [pltpu.VMEM((NSL,), jnp.float32)] * 7
            + [pltpu.VMEM((NSL,), jnp.int32)]
            + [pltpu.VMEM((NSL,), jnp.float32)] * 3
        ),
    )
    discharge_node, thickness = k4(dis_p, vs_p, vc_p, local, stat_p, ob_p)

    k5 = pl.kernel(
        _k5_body, out_type=_f32((E,)), mesh=mesh, compiler_params=cp,
        scratch_types=(
            [pltpu.VMEM((C,), jnp.int32)] * 2
            + [pltpu.VMEM((C,), jnp.float32)] * 9
            + [pltpu.VMEM((1568,), jnp.float32)]
            + [pltpu.VMEM_SHARED((NPAD,), jnp.float32)] * 4
        ),
    )
    residual = k5(head, tail, potential, thickness, discharge_node, inflow)
    return residual


# speedup vs baseline: 272.0952x; 1.1244x over previous
import functools

import jax
import jax.numpy as jnp
from jax import lax
from jax.experimental import pallas as pl
from jax.experimental.pallas import tpu as pltpu
from jax.experimental.pallas import tpu_sc as plsc

RHO_W = 1000.0
RHO_I = 917.0
G = 9.81
SEC_PER_A = 31556926.0
DX = 100.0
CELL_AREA = DX * DX
SHEET_K = 0.01
STEP_H = 0.1
SPACING = 2.0
CLOSURE = 5e-25
NEXP = 3

N = 100000
E = 1600000
K_ADJ = 8

NC = 2
NS = 16
NW = NC * NS
NPAD = 100352
NSL = NPAD // NW
EW = E // NW
C = 2000
NCH = EW // C
L = 16


def _mesh():
    return plsc.VectorSubcoreMesh(core_axis_name="c", subcore_axis_name="s",
                                  num_cores=NC, num_subcores=NS)


def _wid():
    return lax.axis_index("s") * NC + lax.axis_index("c")


def _rsqrt(x):
    i = plsc.bitcast(x, jnp.int32)
    i = 0x5F3759DF - lax.shift_right_logical(i, 1)
    y = plsc.bitcast(i, jnp.float32)
    for _ in range(3):
        y = y * (1.5 - 0.5 * x * y * y)
    return y


def _k1_body(bed, ob, melt, smelt, bp_out, loc_out, pot_out,
             bedv, obv, meltv, smeltv, bpv, locv, potv):
    w = _wid()
    off = w * NSL
    pltpu.sync_copy(bed.at[pl.ds(off, NSL)], bedv)
    pltpu.sync_copy(ob.at[pl.ds(off, NSL)], obv)
    pltpu.sync_copy(melt.at[pl.ds(off, NSL)], meltv)
    pltpu.sync_copy(smelt.at[pl.ds(off, NSL)], smeltv)

    def body(i, _):
        s = pl.ds(i * L, L)
        b = bedv[s]
        o = obv[s]
        bp = RHO_W * G * b + o
        bpv[s] = bp
        potv[s] = bp - o
        locv[s] = (meltv[s] * (RHO_W / RHO_I / SEC_PER_A) + smeltv[s]) * CELL_AREA
        return 0

    lax.fori_loop(0, NSL // L, body, 0)
    pltpu.sync_copy(bpv, bp_out.at[pl.ds(off, NSL)])
    pltpu.sync_copy(locv, loc_out.at[pl.ds(off, NSL)])
    pltpu.sync_copy(potv, pot_out.at[pl.ds(off, NSL)])


def _k2_body(bp, adjt, status, if_out, bptab, adjv, statv, accv, outv):
    w = _wid()
    off = w * NSL
    pltpu.sync_copy(bp, bptab)
    pltpu.sync_copy(status.at[pl.ds(off, NSL)], statv)

    def zero(i, _):
        accv[pl.ds(i * L, L)] = jnp.zeros((L,), jnp.float32)
        return 0

    lax.fori_loop(0, NSL // L, zero, 0)

    def per_j(j, _):
        joff = pl.multiple_of(j * NPAD + off, 8)
        pltpu.sync_copy(adjt.at[pl.ds(joff, NSL)], adjv)

        def per_i(i, _):
            s = pl.ds(i * L, L)
            idx = adjv[s]
            accv[s] = accv[s] + plsc.load_gather(bptab, [idx])
            return 0

        lax.fori_loop(0, NSL // L, per_i, 0)
        return 0

    lax.fori_loop(0, K_ADJ, per_j, 0)

    def fin(i, _):
        s = pl.ds(i * L, L)
        adj_pot = accv[s] * (1.0 / K_ADJ)
        mybp = bptab[pl.ds(off + i * L, L)]
        sign = jnp.where(mybp > adj_pot, 1.0, -1.0)
        outv[s] = jnp.where(statv[s] > 0, sign, 0.0)
        return 0

    lax.fori_loop(0, NSL // L, fin, 0)
    pltpu.sync_copy(outv, if_out.at[pl.ds(off, NSL)])


def _k3_body(head, tail, slide, bp, loc,
             dis_out, vs_out, vc_out,
             headv, tailv, slidev, recvv, sendv, lsendv, onesv, zv,
             bphv, bptv, bouncev, bp_sh, loc_sh, dis_sh, vs_sh, vc_sh):
    c = lax.axis_index("c")
    s = lax.axis_index("s")
    w = s * NC + c
    noff = s * NSL

    def zfill(i, _):
        zv[pl.ds(i * L, L)] = jnp.zeros((L,), jnp.float32)
        onesv[pl.ds(i * L, L)] = jnp.full((L,), 1.0, jnp.float32)
        return 0

    lax.fori_loop(0, C // L, zfill, 0)

    def stripe2(k, _):
        soff = noff + k * 1568
        pltpu.sync_copy(zv.at[pl.ds(0, 1568)], dis_sh.at[pl.ds(soff, 1568)])
        pltpu.sync_copy(zv.at[pl.ds(0, 1568)], vs_sh.at[pl.ds(soff, 1568)])
        pltpu.sync_copy(zv.at[pl.ds(0, 1568)], vc_sh.at[pl.ds(soff, 1568)])
        pltpu.sync_copy(loc.at[pl.ds(soff, 1568)], bouncev)
        pltpu.sync_copy(bouncev, loc_sh.at[pl.ds(soff, 1568)])
        pltpu.sync_copy(bp.at[pl.ds(soff, 1568)], bouncev)
        pltpu.sync_copy(bouncev, bp_sh.at[pl.ds(soff, 1568)])
        return 0

    lax.fori_loop(0, 2, stripe2, 0)
    plsc.subcore_barrier()

    def chunk(ci, _):
        off = w * EW + ci * C
        pltpu.sync_copy(head.at[pl.ds(off, C)], headv)
        pltpu.sync_copy(tail.at[pl.ds(off, C)], tailv)
        pltpu.sync_copy(slide.at[pl.ds(off, C)], slidev)

        pltpu.sync_copy(bp_sh.at[headv], bphv)
        pltpu.sync_copy(bp_sh.at[tailv], bptv)

        def vb(i, _):
            sl = pl.ds(i * L, L)
            h = headv[sl]
            t = tailv[sl]
            down = bptv[sl] > bphv[sl]
            recvv[sl] = jnp.where(down, h, t)
            sendv[sl] = jnp.where(down, t, h)
            return 0

        lax.fori_loop(0, C // L, vb, 0)

        pltpu.sync_copy(loc_sh.at[sendv], lsendv)
        pltpu.sync_copy(lsendv, dis_sh.at[recvv], add=True)
        pltpu.sync_copy(slidev, vs_sh.at[headv], add=True)
        pltpu.sync_copy(slidev, vs_sh.at[tailv], add=True)
        pltpu.sync_copy(onesv, vc_sh.at[headv], add=True)
        pltpu.sync_copy(onesv, vc_sh.at[tailv], add=True)
        return 0

    lax.fori_loop(0, NCH, chunk, 0)
    plsc.subcore_barrier()

    def out2(k, _):
        soff = noff + k * 1568
        hoff = c * NPAD + soff
        pltpu.sync_copy(dis_sh.at[pl.ds(soff, 1568)], bouncev)
        pltpu.sync_copy(bouncev, dis_out.at[pl.ds(hoff, 1568)])
        pltpu.sync_copy(vs_sh.at[pl.ds(soff, 1568)], bouncev)
        pltpu.sync_copy(bouncev, vs_out.at[pl.ds(hoff, 1568)])
        pltpu.sync_copy(vc_sh.at[pl.ds(soff, 1568)], bouncev)
        pltpu.sync_copy(bouncev, vc_out.at[pl.ds(hoff, 1568)])
        return 0

    lax.fori_loop(0, 2, out2, 0)


def _k4_body(dis_p, vs_p, vc_p, loc, status, ob, inf, dn_out, th_out,
             d0v, d1v, v0v, v1v, c0v, c1v, locv, statv, obv, infv, dnv, thv):
    w = _wid()
    off = w * NSL
    pltpu.sync_copy(dis_p.at[pl.ds(off, NSL)], d0v)
    pltpu.sync_copy(dis_p.at[pl.ds(NPAD + off, NSL)], d1v)
    pltpu.sync_copy(vs_p.at[pl.ds(off, NSL)], v0v)
    pltpu.sync_copy(vs_p.at[pl.ds(NPAD + off, NSL)], v1v)
    pltpu.sync_copy(vc_p.at[pl.ds(off, NSL)], c0v)
    pltpu.sync_copy(vc_p.at[pl.ds(NPAD + off, NSL)], c1v)
    pltpu.sync_copy(loc.at[pl.ds(off, NSL)], locv)
    pltpu.sync_copy(status.at[pl.ds(off, NSL)], statv)
    pltpu.sync_copy(ob.at[pl.ds(off, NSL)], obv)
    pltpu.sync_copy(inf.at[pl.ds(off, NSL)], infv)

    def body(i, _):
        s = pl.ds(i * L, L)
        dn = locv[s] + d0v[s] + d1v[s]
        dn = jnp.where(statv[s] != 0, 0.0, dn)
        dbits = plsc.bitcast(dn, jnp.int32)
        dbits = jnp.where(infv[s] > 0.5, dbits | jnp.int32(-2147483648), dbits)
        dnv[s] = plsc.bitcast(dbits, jnp.float32)
        vsum = v0v[s] + v1v[s]
        vcnt = c0v[s] + c1v[s]
        sn = jnp.abs(vsum / jnp.maximum(vcnt, 1.0)) * (1.0 / SEC_PER_A)
        p = obv[s]
        thv[s] = sn * sn * STEP_H / (CLOSURE * p * p * p * (SPACING * SPACING) + 1e-30)
        return 0

    lax.fori_loop(0, NSL // L, body, 0)
    pltpu.sync_copy(dnv, dn_out.at[pl.ds(off, NSL)])
    pltpu.sync_copy(thv, th_out.at[pl.ds(off, NSL)])


def _k5_body(head, tail, pot, th, dn, res_out,
             headv, tailv, phv, ptv, thv, ttv, dhv, dtv, outv,
             bouncev, pot_sh, th_sh, dn_sh):
    c = lax.axis_index("c")
    s = lax.axis_index("s")
    w = s * NC + c
    noff = s * NSL

    def tload(k, _):
        soff = noff + k * 1568
        for hbm_ref, sh_ref in ((pot, pot_sh), (th, th_sh), (dn, dn_sh)):
            pltpu.sync_copy(hbm_ref.at[pl.ds(soff, 1568)], bouncev)
            pltpu.sync_copy(bouncev, sh_ref.at[pl.ds(soff, 1568)])
        return 0

    lax.fori_loop(0, 2, tload, 0)
    plsc.subcore_barrier()

    def chunk(ci, _):
        off = w * EW + ci * C
        pltpu.sync_copy(head.at[pl.ds(off, C)], headv)
        pltpu.sync_copy(tail.at[pl.ds(off, C)], tailv)
        pltpu.sync_copy(pot_sh.at[headv], phv)
        pltpu.sync_copy(pot_sh.at[tailv], ptv)
        pltpu.sync_copy(th_sh.at[headv], thv)
        pltpu.sync_copy(th_sh.at[tailv], ttv)
        pltpu.sync_copy(dn_sh.at[headv], dhv)
        pltpu.sync_copy(dn_sh.at[tailv], dtv)

        def vb(i, _):
            sl = pl.ds(i * L, L)
            hl = 0.5 * (thv[sl] + ttv[sl])
            g = (phv[sl] - ptv[sl]) * (1.0 / DX)
            a = jnp.abs(g) + 1e-12
            r = _rsqrt(a)
            q = _rsqrt(_rsqrt(hl))
            flux = (-SHEET_K) * hl * q * r * g
            dhb = plsc.bitcast(dhv[sl], jnp.int32)
            dtb = plsc.bitcast(dtv[sl], jnp.int32)
            gate = (dhb < 0) | (dtb < 0)
            flux = jnp.where(gate, 0.0, flux)
            d = 0.5 * (jnp.abs(dhv[sl]) + jnp.abs(dtv[sl]))
            outv[sl] = jnp.abs(flux - d)
            return 0

        lax.fori_loop(0, C // L, vb, 0)
        pltpu.sync_copy(outv, res_out.at[pl.ds(off, C)])
        return 0

    lax.fori_loop(0, NCH, chunk, 0)


def _f32(shape):
    return jax.ShapeDtypeStruct(shape, jnp.float32)


def kernel(edge_index, adjacent_nodes, status_at_node, bedrock_elevation,
           overburden_pressure, melt_rate, surface_melt_rate, sliding_velocity):
    head = edge_index[0]
    tail = edge_index[1]
    pad = NPAD - N
    bed_p = jnp.pad(bedrock_elevation, (0, pad))
    ob_p = jnp.pad(overburden_pressure, (0, pad))
    melt_p = jnp.pad(melt_rate, (0, pad))
    smelt_p = jnp.pad(surface_melt_rate, (0, pad))
    stat_p = jnp.pad(status_at_node, (0, pad))
    adjt_p = jnp.pad(adjacent_nodes.T, ((0, 0), (0, pad))).reshape(-1)

    mesh = _mesh()
    cp = pltpu.CompilerParams(needs_layout_passes=False)

    k1 = pl.kernel(
        _k1_body, out_type=(_f32((NPAD,)),) * 3, mesh=mesh, compiler_params=cp,
        scratch_types=[pltpu.VMEM((NSL,), jnp.float32)] * 7,
    )
    base_pot, local, potential = k1(bed_p, ob_p, melt_p, smelt_p)

    k2 = pl.kernel(
        _k2_body, out_type=_f32((NPAD,)), mesh=mesh, compiler_params=cp,
        scratch_types=[
            pltpu.VMEM((NPAD,), jnp.float32),
            pltpu.VMEM((NSL,), jnp.int32),
            pltpu.VMEM((NSL,), jnp.int32),
            pltpu.VMEM((NSL,), jnp.float32),
            pltpu.VMEM((NSL,), jnp.float32),
        ],
    )
    inflow = k2(base_pot, adjt_p, stat_p)

    k3 = pl.kernel(
        _k3_body, out_type=(_f32((NC * NPAD,)),) * 3, mesh=mesh, compiler_params=cp,
        scratch_types=[
            pltpu.VMEM((C,), jnp.int32),
            pltpu.VMEM((C,), jnp.int32),
            pltpu.VMEM((C,), jnp.float32),
            pltpu.VMEM((C,), jnp.int32),
            pltpu.VMEM((C,), jnp.int32),
            pltpu.VMEM((C,), jnp.float32),
            pltpu.VMEM((C,), jnp.float32),
            pltpu.VMEM((C,), jnp.float32),
            pltpu.VMEM((C,), jnp.float32),
            pltpu.VMEM((C,), jnp.float32),
            pltpu.VMEM((1568,), jnp.float32),
            pltpu.VMEM_SHARED((NPAD,), jnp.float32),
            pltpu.VMEM_SHARED((NPAD,), jnp.float32),
            pltpu.VMEM_SHARED((NPAD,), jnp.float32),
            pltpu.VMEM_SHARED((NPAD,), jnp.float32),
            pltpu.VMEM_SHARED((NPAD,), jnp.float32),
        ],
    )
    dis_p, vs_p, vc_p = k3(head, tail, sliding_velocity, base_pot, local)

    k4 = pl.kernel(
        _k4_body, out_type=(_f32((NPAD,)),) * 2, mesh=mesh, compiler_params=cp,
        scratch_types=(
            [pltpu.VMEM((NSL,), jnp.float32)] * 7
            + [pltpu.VMEM((NSL,), jnp.int32)]
            + [pltpu.VMEM((NSL,), jnp.float32)] * 4
        ),
    )
    discharge_node, thickness = k4(dis_p, vs_p, vc_p, local, stat_p, ob_p, inflow)

    k5 = pl.kernel(
        _k5_body, out_type=_f32((E,)), mesh=mesh, compiler_params=cp,
        scratch_types=(
            [pltpu.VMEM((C,), jnp.int32)] * 2
            + [pltpu.VMEM((C,), jnp.float32)] * 7
            + [pltpu.VMEM((1568,), jnp.float32)]
            + [pltpu.VMEM_SHARED((NPAD,), jnp.float32)] * 3
        ),
    )
    residual = k5(head, tail, potential, thickness, discharge_node)
    return residual


# speedup vs baseline: 279.1058x; 1.0258x over previous
import functools

import jax
import jax.numpy as jnp
from jax import lax
from jax.experimental import pallas as pl
from jax.experimental.pallas import tpu as pltpu
from jax.experimental.pallas import tpu_sc as plsc

RHO_W = 1000.0
RHO_I = 917.0
G = 9.81
SEC_PER_A = 31556926.0
DX = 100.0
CELL_AREA = DX * DX
SHEET_K = 0.01
STEP_H = 0.1
SPACING = 2.0
CLOSURE = 5e-25
NEXP = 3

N = 100000
E = 1600000
K_ADJ = 8

NC = 2
NS = 16
NW = NC * NS
NPAD = 100352
NSL = NPAD // NW
EW = E // NW
C = 2000
NCH = EW // C
L = 16


def _mesh():
    return plsc.VectorSubcoreMesh(core_axis_name="c", subcore_axis_name="s",
                                  num_cores=NC, num_subcores=NS)


def _wid():
    return lax.axis_index("s") * NC + lax.axis_index("c")


def _rsqrt(x):
    i = plsc.bitcast(x, jnp.int32)
    i = 0x5F3759DF - lax.shift_right_logical(i, 1)
    y = plsc.bitcast(i, jnp.float32)
    for _ in range(3):
        y = y * (1.5 - 0.5 * x * y * y)
    return y


def _k1_body(bed, ob, melt, smelt, bp_out, loc_out, pot_out,
             bedv, obv, meltv, smeltv, bpv, locv, potv):
    w = _wid()
    off = w * NSL
    pltpu.sync_copy(bed.at[pl.ds(off, NSL)], bedv)
    pltpu.sync_copy(ob.at[pl.ds(off, NSL)], obv)
    pltpu.sync_copy(melt.at[pl.ds(off, NSL)], meltv)
    pltpu.sync_copy(smelt.at[pl.ds(off, NSL)], smeltv)

    def body(i, _):
        s = pl.ds(i * L, L)
        b = bedv[s]
        o = obv[s]
        bp = RHO_W * G * b + o
        bpv[s] = bp
        potv[s] = bp - o
        locv[s] = (meltv[s] * (RHO_W / RHO_I / SEC_PER_A) + smeltv[s]) * CELL_AREA
        return 0

    lax.fori_loop(0, NSL // L, body, 0)
    pltpu.sync_copy(bpv, bp_out.at[pl.ds(off, NSL)])
    pltpu.sync_copy(locv, loc_out.at[pl.ds(off, NSL)])
    pltpu.sync_copy(potv, pot_out.at[pl.ds(off, NSL)])


def _k2_body(bp, adjt, status, if_out, bptab, adjv, statv, accv, outv):
    w = _wid()
    off = w * NSL
    pltpu.sync_copy(bp, bptab)
    pltpu.sync_copy(status.at[pl.ds(off, NSL)], statv)

    def zero(i, _):
        accv[pl.ds(i * L, L)] = jnp.zeros((L,), jnp.float32)
        return 0

    lax.fori_loop(0, NSL // L, zero, 0)

    def per_j(j, _):
        joff = pl.multiple_of(j * NPAD + off, 8)
        pltpu.sync_copy(adjt.at[pl.ds(joff, NSL)], adjv)

        def per_i(i, _):
            s = pl.ds(i * L, L)
            idx = adjv[s]
            accv[s] = accv[s] + plsc.load_gather(bptab, [idx])
            return 0

        lax.fori_loop(0, NSL // L, per_i, 0)
        return 0

    lax.fori_loop(0, K_ADJ, per_j, 0)

    def fin(i, _):
        s = pl.ds(i * L, L)
        adj_pot = accv[s] * (1.0 / K_ADJ)
        mybp = bptab[pl.ds(off + i * L, L)]
        sign = jnp.where(mybp > adj_pot, 1.0, -1.0)
        outv[s] = jnp.where(statv[s] > 0, sign, 0.0)
        return 0

    lax.fori_loop(0, NSL // L, fin, 0)
    pltpu.sync_copy(outv, if_out.at[pl.ds(off, NSL)])


def _k3_body(head, tail, slide, bp, loc,
             dis_out, vs_out, vc_out,
             headv, tailv, slidev, recvv, sendv, lsendv, onesv, zv,
             bphv, bptv, bouncev, bp_sh, loc_sh, dis_sh, vs_sh, vc_sh):
    c = lax.axis_index("c")
    s = lax.axis_index("s")
    w = s * NC + c
    noff = s * NSL

    def zfill(i, _):
        zv[pl.ds(i * L, L)] = jnp.zeros((L,), jnp.float32)
        onesv[pl.ds(i * L, L)] = jnp.full((L,), 1.0, jnp.float32)
        return 0

    lax.fori_loop(0, C // L, zfill, 0)

    def stripe2(k, _):
        soff = noff + k * 1568
        pltpu.sync_copy(zv.at[pl.ds(0, 1568)], dis_sh.at[pl.ds(soff, 1568)])
        pltpu.sync_copy(zv.at[pl.ds(0, 1568)], vs_sh.at[pl.ds(soff, 1568)])
        pltpu.sync_copy(zv.at[pl.ds(0, 1568)], vc_sh.at[pl.ds(soff, 1568)])
        pltpu.sync_copy(loc.at[pl.ds(soff, 1568)], bouncev)
        pltpu.sync_copy(bouncev, loc_sh.at[pl.ds(soff, 1568)])
        pltpu.sync_copy(bp.at[pl.ds(soff, 1568)], bouncev)
        pltpu.sync_copy(bouncev, bp_sh.at[pl.ds(soff, 1568)])
        return 0

    lax.fori_loop(0, 2, stripe2, 0)
    plsc.subcore_barrier()

    def chunk(ci, _):
        off = w * EW + ci * C
        pltpu.sync_copy(head.at[pl.ds(off, C)], headv)
        pltpu.sync_copy(tail.at[pl.ds(off, C)], tailv)
        pltpu.sync_copy(slide.at[pl.ds(off, C)], slidev)

        pltpu.sync_copy(bp_sh.at[headv], bphv)
        pltpu.sync_copy(bp_sh.at[tailv], bptv)

        def vb(i, _):
            sl = pl.ds(i * L, L)
            h = headv[sl]
            t = tailv[sl]
            down = bptv[sl] > bphv[sl]
            recvv[sl] = jnp.where(down, h, t)
            sendv[sl] = jnp.where(down, t, h)
            return 0

        lax.fori_loop(0, C // L, vb, 0)

        pltpu.sync_copy(loc_sh.at[sendv], lsendv)
        pltpu.sync_copy(lsendv, dis_sh.at[recvv], add=True)
        pltpu.sync_copy(slidev, vs_sh.at[headv], add=True)
        pltpu.sync_copy(slidev, vs_sh.at[tailv], add=True)
        pltpu.sync_copy(onesv, vc_sh.at[headv], add=True)
        pltpu.sync_copy(onesv, vc_sh.at[tailv], add=True)
        return 0

    lax.fori_loop(0, NCH, chunk, 0)
    plsc.subcore_barrier()

    def out2(k, _):
        soff = noff + k * 1568
        hoff = c * NPAD + soff
        pltpu.sync_copy(dis_sh.at[pl.ds(soff, 1568)], bouncev)
        pltpu.sync_copy(bouncev, dis_out.at[pl.ds(hoff, 1568)])
        pltpu.sync_copy(vs_sh.at[pl.ds(soff, 1568)], bouncev)
        pltpu.sync_copy(bouncev, vs_out.at[pl.ds(hoff, 1568)])
        pltpu.sync_copy(vc_sh.at[pl.ds(soff, 1568)], bouncev)
        pltpu.sync_copy(bouncev, vc_out.at[pl.ds(hoff, 1568)])
        return 0

    lax.fori_loop(0, 2, out2, 0)


def _k4_body(dis_p, vs_p, vc_p, loc, status, ob, inf, dn_out, th_out,
             d0v, d1v, v0v, v1v, c0v, c1v, locv, statv, obv, infv, dnv, thv):
    w = _wid()
    off = w * NSL
    pltpu.sync_copy(dis_p.at[pl.ds(off, NSL)], d0v)
    pltpu.sync_copy(dis_p.at[pl.ds(NPAD + off, NSL)], d1v)
    pltpu.sync_copy(vs_p.at[pl.ds(off, NSL)], v0v)
    pltpu.sync_copy(vs_p.at[pl.ds(NPAD + off, NSL)], v1v)
    pltpu.sync_copy(vc_p.at[pl.ds(off, NSL)], c0v)
    pltpu.sync_copy(vc_p.at[pl.ds(NPAD + off, NSL)], c1v)
    pltpu.sync_copy(loc.at[pl.ds(off, NSL)], locv)
    pltpu.sync_copy(status.at[pl.ds(off, NSL)], statv)
    pltpu.sync_copy(ob.at[pl.ds(off, NSL)], obv)
    pltpu.sync_copy(inf.at[pl.ds(off, NSL)], infv)

    def body(i, _):
        s = pl.ds(i * L, L)
        dn = locv[s] + d0v[s] + d1v[s]
        dn = jnp.where(statv[s] != 0, 0.0, dn)
        dbits = plsc.bitcast(dn, jnp.int32)
        dbits = jnp.where(infv[s] > 0.5, dbits | jnp.int32(-2147483648), dbits)
        dnv[s] = plsc.bitcast(dbits, jnp.float32)
        vsum = v0v[s] + v1v[s]
        vcnt = c0v[s] + c1v[s]
        sn = jnp.abs(vsum / jnp.maximum(vcnt, 1.0)) * (1.0 / SEC_PER_A)
        p = obv[s]
        thv[s] = sn * sn * STEP_H / (CLOSURE * p * p * p * (SPACING * SPACING) + 1e-30)
        return 0

    lax.fori_loop(0, NSL // L, body, 0)
    pltpu.sync_copy(dnv, dn_out.at[pl.ds(off, NSL)])
    pltpu.sync_copy(thv, th_out.at[pl.ds(off, NSL)])


def _k5_body(head, tail, pot, th, dn, res_out,
             headv, tailv, phv, ptv, thv, ttv, dhv, dtv, outv,
             bouncev, sem, pot_sh, th_sh, dn_sh):
    c = lax.axis_index("c")
    s = lax.axis_index("s")
    w = s * NC + c
    noff = s * NSL

    def tload(k, _):
        soff = noff + k * 1568
        for hbm_ref, sh_ref in ((pot, pot_sh), (th, th_sh), (dn, dn_sh)):
            pltpu.sync_copy(hbm_ref.at[pl.ds(soff, 1568)], bouncev)
            pltpu.sync_copy(bouncev, sh_ref.at[pl.ds(soff, 1568)])
        return 0

    lax.fori_loop(0, 2, tload, 0)
    plsc.subcore_barrier()

    def chunk(ci, _):
        off = w * EW + ci * C
        pltpu.sync_copy(head.at[pl.ds(off, C)], headv)
        pltpu.sync_copy(tail.at[pl.ds(off, C)], tailv)
        cps = [
            pltpu.async_copy(pot_sh.at[headv], phv, sem),
            pltpu.async_copy(pot_sh.at[tailv], ptv, sem),
            pltpu.async_copy(th_sh.at[headv], thv, sem),
            pltpu.async_copy(th_sh.at[tailv], ttv, sem),
            pltpu.async_copy(dn_sh.at[headv], dhv, sem),
            pltpu.async_copy(dn_sh.at[tailv], dtv, sem),
        ]
        for cp_ in cps:
            cp_.wait()

        def vb(i, _):
            sl = pl.ds(i * L, L)
            hl = 0.5 * (thv[sl] + ttv[sl])
            g = (phv[sl] - ptv[sl]) * (1.0 / DX)
            a = jnp.abs(g) + 1e-12
            r = _rsqrt(a)
            q = _rsqrt(_rsqrt(hl))
            flux = (-SHEET_K) * hl * q * r * g
            dhb = plsc.bitcast(dhv[sl], jnp.int32)
            dtb = plsc.bitcast(dtv[sl], jnp.int32)
            gate = (dhb < 0) | (dtb < 0)
            flux = jnp.where(gate, 0.0, flux)
            d = 0.5 * (jnp.abs(dhv[sl]) + jnp.abs(dtv[sl]))
            outv[sl] = jnp.abs(flux - d)
            return 0

        lax.fori_loop(0, C // L, vb, 0)
        pltpu.sync_copy(outv, res_out.at[pl.ds(off, C)])
        return 0

    lax.fori_loop(0, NCH, chunk, 0)


def _f32(shape):
    return jax.ShapeDtypeStruct(shape, jnp.float32)


def kernel(edge_index, adjacent_nodes, status_at_node, bedrock_elevation,
           overburden_pressure, melt_rate, surface_melt_rate, sliding_velocity):
    head = edge_index[0]
    tail = edge_index[1]
    pad = NPAD - N
    bed_p = jnp.pad(bedrock_elevation, (0, pad))
    ob_p = jnp.pad(overburden_pressure, (0, pad))
    melt_p = jnp.pad(melt_rate, (0, pad))
    smelt_p = jnp.pad(surface_melt_rate, (0, pad))
    stat_p = jnp.pad(status_at_node, (0, pad))
    adjt_p = jnp.pad(adjacent_nodes.T, ((0, 0), (0, pad))).reshape(-1)

    mesh = _mesh()
    cp = pltpu.CompilerParams(needs_layout_passes=False)

    k1 = pl.kernel(
        _k1_body, out_type=(_f32((NPAD,)),) * 3, mesh=mesh, compiler_params=cp,
        scratch_types=[pltpu.VMEM((NSL,), jnp.float32)] * 7,
    )
    base_pot, local, potential = k1(bed_p, ob_p, melt_p, smelt_p)

    k2 = pl.kernel(
        _k2_body, out_type=_f32((NPAD,)), mesh=mesh, compiler_params=cp,
        scratch_types=[
            pltpu.VMEM((NPAD,), jnp.float32),
            pltpu.VMEM((NSL,), jnp.int32),
            pltpu.VMEM((NSL,), jnp.int32),
            pltpu.VMEM((NSL,), jnp.float32),
            pltpu.VMEM((NSL,), jnp.float32),
        ],
    )
    inflow = k2(base_pot, adjt_p, stat_p)

    k3 = pl.kernel(
        _k3_body, out_type=(_f32((NC * NPAD,)),) * 3, mesh=mesh, compiler_params=cp,
        scratch_types=[
            pltpu.VMEM((C,), jnp.int32),
            pltpu.VMEM((C,), jnp.int32),
            pltpu.VMEM((C,), jnp.float32),
            pltpu.VMEM((C,), jnp.int32),
            pltpu.VMEM((C,), jnp.int32),
            pltpu.VMEM((C,), jnp.float32),
            pltpu.VMEM((C,), jnp.float32),
            pltpu.VMEM((C,), jnp.float32),
            pltpu.VMEM((C,), jnp.float32),
            pltpu.VMEM((C,), jnp.float32),
            pltpu.VMEM((1568,), jnp.float32),
            pltpu.VMEM_SHARED((NPAD,), jnp.float32),
            pltpu.VMEM_SHARED((NPAD,), jnp.float32),
            pltpu.VMEM_SHARED((NPAD,), jnp.float32),
            pltpu.VMEM_SHARED((NPAD,), jnp.float32),
            pltpu.VMEM_SHARED((NPAD,), jnp.float32),
        ],
    )
    dis_p, vs_p, vc_p = k3(head, tail, sliding_velocity, base_pot, local)

    k4 = pl.kernel(
        _k4_body, out_type=(_f32((NPAD,)),) * 2, mesh=mesh, compiler_params=cp,
        scratch_types=(
            [pltpu.VMEM((NSL,), jnp.float32)] * 7
            + [pltpu.VMEM((NSL,), jnp.int32)]
            + [pltpu.VMEM((NSL,), jnp.float32)] * 4
        ),
    )
    discharge_node, thickness = k4(dis_p, vs_p, vc_p, local, stat_p, ob_p, inflow)

    k5 = pl.kernel(
        _k5_body, out_type=_f32((E,)), mesh=mesh, compiler_params=cp,
        scratch_types=(
            [pltpu.VMEM((C,), jnp.int32)] * 2
            + [pltpu.VMEM((C,), jnp.float32)] * 7
            + [pltpu.VMEM((1568,), jnp.float32)]
            + [pltpu.SemaphoreType.DMA]
            + [pltpu.VMEM_SHARED((NPAD,), jnp.float32)] * 3
        ),
    )
    residual = k5(head, tail, potential, thickness, discharge_node)
    return residual


# speedup vs baseline: 310.4528x; 1.1123x over previous
import functools

import jax
import jax.numpy as jnp
from jax import lax
from jax.experimental import pallas as pl
from jax.experimental.pallas import tpu as pltpu
from jax.experimental.pallas import tpu_sc as plsc

RHO_W = 1000.0
RHO_I = 917.0
G = 9.81
SEC_PER_A = 31556926.0
DX = 100.0
CELL_AREA = DX * DX
SHEET_K = 0.01
STEP_H = 0.1
SPACING = 2.0
CLOSURE = 5e-25
NEXP = 3

N = 100000
E = 1600000
K_ADJ = 8

NC = 2
NS = 16
NW = NC * NS
NPAD = 100352
NSL = NPAD // NW
EW = E // NW
C = 2000
NCH = EW // C
L = 16


def _mesh():
    return plsc.VectorSubcoreMesh(core_axis_name="c", subcore_axis_name="s",
                                  num_cores=NC, num_subcores=NS)


def _wid():
    return lax.axis_index("s") * NC + lax.axis_index("c")


def _rsqrt(x):
    i = plsc.bitcast(x, jnp.int32)
    i = 0x5F3759DF - lax.shift_right_logical(i, 1)
    y = plsc.bitcast(i, jnp.float32)
    for _ in range(3):
        y = y * (1.5 - 0.5 * x * y * y)
    return y


def _k1_body(bed, ob, melt, smelt, bp_out, loc_out, pot_out,
             bedv, obv, meltv, smeltv, bpv, locv, potv):
    w = _wid()
    off = w * NSL
    pltpu.sync_copy(bed.at[pl.ds(off, NSL)], bedv)
    pltpu.sync_copy(ob.at[pl.ds(off, NSL)], obv)
    pltpu.sync_copy(melt.at[pl.ds(off, NSL)], meltv)
    pltpu.sync_copy(smelt.at[pl.ds(off, NSL)], smeltv)

    def body(i, _):
        s = pl.ds(i * L, L)
        b = bedv[s]
        o = obv[s]
        bp = RHO_W * G * b + o
        bpv[s] = bp
        potv[s] = bp - o
        locv[s] = (meltv[s] * (RHO_W / RHO_I / SEC_PER_A) + smeltv[s]) * CELL_AREA
        return 0

    lax.fori_loop(0, NSL // L, body, 0)
    pltpu.sync_copy(bpv, bp_out.at[pl.ds(off, NSL)])
    pltpu.sync_copy(locv, loc_out.at[pl.ds(off, NSL)])
    pltpu.sync_copy(potv, pot_out.at[pl.ds(off, NSL)])


def _k2_body(bp, adjt, status, if_out, bptab, adjv, statv, accv, outv):
    w = _wid()
    off = w * NSL
    pltpu.sync_copy(bp, bptab)
    pltpu.sync_copy(status.at[pl.ds(off, NSL)], statv)

    def zero(i, _):
        accv[pl.ds(i * L, L)] = jnp.zeros((L,), jnp.float32)
        return 0

    lax.fori_loop(0, NSL // L, zero, 0)

    def per_j(j, _):
        joff = pl.multiple_of(j * NPAD + off, 8)
        pltpu.sync_copy(adjt.at[pl.ds(joff, NSL)], adjv)

        def per_i(i, _):
            s = pl.ds(i * L, L)
            idx = adjv[s]
            accv[s] = accv[s] + plsc.load_gather(bptab, [idx])
            return 0

        lax.fori_loop(0, NSL // L, per_i, 0)
        return 0

    lax.fori_loop(0, K_ADJ, per_j, 0)

    def fin(i, _):
        s = pl.ds(i * L, L)
        adj_pot = accv[s] * (1.0 / K_ADJ)
        mybp = bptab[pl.ds(off + i * L, L)]
        sign = jnp.where(mybp > adj_pot, 1.0, -1.0)
        outv[s] = jnp.where(statv[s] > 0, sign, 0.0)
        return 0

    lax.fori_loop(0, NSL // L, fin, 0)
    pltpu.sync_copy(outv, if_out.at[pl.ds(off, NSL)])


def _k3_body(head, tail, slide, bp, loc,
             dis_out, vs_out, vc_out,
             headv, tailv, slidev, recvv, sendv, lsendv, onesv, zv,
             bphv, bptv, bouncev, sem, ssem, bp_sh, loc_sh, dis_sh, vs_sh, vc_sh):
    c = lax.axis_index("c")
    s = lax.axis_index("s")
    w = s * NC + c
    noff = s * NSL

    def zfill(i, _):
        zv[pl.ds(i * L, L)] = jnp.zeros((L,), jnp.float32)
        onesv[pl.ds(i * L, L)] = jnp.full((L,), 1.0, jnp.float32)
        return 0

    lax.fori_loop(0, C // L, zfill, 0)

    def stripe2(k, _):
        soff = noff + k * 1568
        pltpu.sync_copy(zv.at[pl.ds(0, 1568)], dis_sh.at[pl.ds(soff, 1568)])
        pltpu.sync_copy(zv.at[pl.ds(0, 1568)], vs_sh.at[pl.ds(soff, 1568)])
        pltpu.sync_copy(zv.at[pl.ds(0, 1568)], vc_sh.at[pl.ds(soff, 1568)])
        pltpu.sync_copy(loc.at[pl.ds(soff, 1568)], bouncev)
        pltpu.sync_copy(bouncev, loc_sh.at[pl.ds(soff, 1568)])
        pltpu.sync_copy(bp.at[pl.ds(soff, 1568)], bouncev)
        pltpu.sync_copy(bouncev, bp_sh.at[pl.ds(soff, 1568)])
        return 0

    lax.fori_loop(0, 2, stripe2, 0)
    plsc.subcore_barrier()

    def chunk(ci, _):
        off = w * EW + ci * C
        lds = [
            pltpu.async_copy(head.at[pl.ds(off, C)], headv, sem),
            pltpu.async_copy(tail.at[pl.ds(off, C)], tailv, sem),
            pltpu.async_copy(slide.at[pl.ds(off, C)], slidev, sem),
        ]
        for ld_ in lds:
            ld_.wait()
        gs = [
            pltpu.async_copy(bp_sh.at[headv], bphv, sem),
            pltpu.async_copy(bp_sh.at[tailv], bptv, sem),
        ]
        for g_ in gs:
            g_.wait()

        def vb(i, _):
            sl = pl.ds(i * L, L)
            h = headv[sl]
            t = tailv[sl]
            down = bptv[sl] > bphv[sl]
            recvv[sl] = jnp.where(down, h, t)
            sendv[sl] = jnp.where(down, t, h)
            return 0

        lax.fori_loop(0, C // L, vb, 0)

        pltpu.sync_copy(loc_sh.at[sendv], lsendv)
        scs = [
            pltpu.async_copy(lsendv, dis_sh.at[recvv], ssem, add=True),
            pltpu.async_copy(slidev, vs_sh.at[headv], ssem, add=True),
            pltpu.async_copy(slidev, vs_sh.at[tailv], ssem, add=True),
            pltpu.async_copy(onesv, vc_sh.at[headv], ssem, add=True),
            pltpu.async_copy(onesv, vc_sh.at[tailv], ssem, add=True),
        ]
        for sc_ in scs:
            sc_.wait()
        return 0

    lax.fori_loop(0, NCH, chunk, 0)
    plsc.subcore_barrier()

    def out2(k, _):
        soff = noff + k * 1568
        hoff = c * NPAD + soff
        pltpu.sync_copy(dis_sh.at[pl.ds(soff, 1568)], bouncev)
        pltpu.sync_copy(bouncev, dis_out.at[pl.ds(hoff, 1568)])
        pltpu.sync_copy(vs_sh.at[pl.ds(soff, 1568)], bouncev)
        pltpu.sync_copy(bouncev, vs_out.at[pl.ds(hoff, 1568)])
        pltpu.sync_copy(vc_sh.at[pl.ds(soff, 1568)], bouncev)
        pltpu.sync_copy(bouncev, vc_out.at[pl.ds(hoff, 1568)])
        return 0

    lax.fori_loop(0, 2, out2, 0)


def _k4_body(dis_p, vs_p, vc_p, loc, status, ob, inf, dn_out, th_out,
             d0v, d1v, v0v, v1v, c0v, c1v, locv, statv, obv, infv, dnv, thv):
    w = _wid()
    off = w * NSL
    pltpu.sync_copy(dis_p.at[pl.ds(off, NSL)], d0v)
    pltpu.sync_copy(dis_p.at[pl.ds(NPAD + off, NSL)], d1v)
    pltpu.sync_copy(vs_p.at[pl.ds(off, NSL)], v0v)
    pltpu.sync_copy(vs_p.at[pl.ds(NPAD + off, NSL)], v1v)
    pltpu.sync_copy(vc_p.at[pl.ds(off, NSL)], c0v)
    pltpu.sync_copy(vc_p.at[pl.ds(NPAD + off, NSL)], c1v)
    pltpu.sync_copy(loc.at[pl.ds(off, NSL)], locv)
    pltpu.sync_copy(status.at[pl.ds(off, NSL)], statv)
    pltpu.sync_copy(ob.at[pl.ds(off, NSL)], obv)
    pltpu.sync_copy(inf.at[pl.ds(off, NSL)], infv)

    def body(i, _):
        s = pl.ds(i * L, L)
        dn = locv[s] + d0v[s] + d1v[s]
        dn = jnp.where(statv[s] != 0, 0.0, dn)
        dbits = plsc.bitcast(dn, jnp.int32)
        dbits = jnp.where(infv[s] > 0.5, dbits | jnp.int32(-2147483648), dbits)
        dnv[s] = plsc.bitcast(dbits, jnp.float32)
        vsum = v0v[s] + v1v[s]
        vcnt = c0v[s] + c1v[s]
        sn = jnp.abs(vsum / jnp.maximum(vcnt, 1.0)) * (1.0 / SEC_PER_A)
        p = obv[s]
        thv[s] = sn * sn * STEP_H / (CLOSURE * p * p * p * (SPACING * SPACING) + 1e-30)
        return 0

    lax.fori_loop(0, NSL // L, body, 0)
    pltpu.sync_copy(dnv, dn_out.at[pl.ds(off, NSL)])
    pltpu.sync_copy(thv, th_out.at[pl.ds(off, NSL)])


def _k5_body(head, tail, pot, th, dn, res_out,
             headv, tailv, phv, ptv, thv, ttv, dhv, dtv, outv,
             bouncev, sem, pot_sh, th_sh, dn_sh):
    c = lax.axis_index("c")
    s = lax.axis_index("s")
    w = s * NC + c
    noff = s * NSL

    def tload(k, _):
        soff = noff + k * 1568
        for hbm_ref, sh_ref in ((pot, pot_sh), (th, th_sh), (dn, dn_sh)):
            pltpu.sync_copy(hbm_ref.at[pl.ds(soff, 1568)], bouncev)
            pltpu.sync_copy(bouncev, sh_ref.at[pl.ds(soff, 1568)])
        return 0

    lax.fori_loop(0, 2, tload, 0)
    plsc.subcore_barrier()

    def chunk(ci, _):
        off = w * EW + ci * C
        lds = [
            pltpu.async_copy(head.at[pl.ds(off, C)], headv, sem),
            pltpu.async_copy(tail.at[pl.ds(off, C)], tailv, sem),
        ]
        for ld_ in lds:
            ld_.wait()
        cps = [
            pltpu.async_copy(pot_sh.at[headv], phv, sem),
            pltpu.async_copy(pot_sh.at[tailv], ptv, sem),
            pltpu.async_copy(th_sh.at[headv], thv, sem),
            pltpu.async_copy(th_sh.at[tailv], ttv, sem),
            pltpu.async_copy(dn_sh.at[headv], dhv, sem),
            pltpu.async_copy(dn_sh.at[tailv], dtv, sem),
        ]
        for cp_ in cps:
            cp_.wait()

        def vb(i, _):
            sl = pl.ds(i * L, L)
            hl = 0.5 * (thv[sl] + ttv[sl])
            g = (phv[sl] - ptv[sl]) * (1.0 / DX)
            a = jnp.abs(g) + 1e-12
            r = _rsqrt(a)
            q = _rsqrt(_rsqrt(hl))
            flux = (-SHEET_K) * hl * q * r * g
            dhb = plsc.bitcast(dhv[sl], jnp.int32)
            dtb = plsc.bitcast(dtv[sl], jnp.int32)
            gate = (dhb < 0) | (dtb < 0)
            flux = jnp.where(gate, 0.0, flux)
            d = 0.5 * (jnp.abs(dhv[sl]) + jnp.abs(dtv[sl]))
            outv[sl] = jnp.abs(flux - d)
            return 0

        lax.fori_loop(0, C // L, vb, 0)
        pltpu.sync_copy(outv, res_out.at[pl.ds(off, C)])
        return 0

    lax.fori_loop(0, NCH, chunk, 0)


def _f32(shape):
    return jax.ShapeDtypeStruct(shape, jnp.float32)


def kernel(edge_index, adjacent_nodes, status_at_node, bedrock_elevation,
           overburden_pressure, melt_rate, surface_melt_rate, sliding_velocity):
    head = edge_index[0]
    tail = edge_index[1]
    pad = NPAD - N
    bed_p = jnp.pad(bedrock_elevation, (0, pad))
    ob_p = jnp.pad(overburden_pressure, (0, pad))
    melt_p = jnp.pad(melt_rate, (0, pad))
    smelt_p = jnp.pad(surface_melt_rate, (0, pad))
    stat_p = jnp.pad(status_at_node, (0, pad))
    adjt_p = jnp.pad(adjacent_nodes.T, ((0, 0), (0, pad))).reshape(-1)

    mesh = _mesh()
    cp = pltpu.CompilerParams(needs_layout_passes=False)

    k1 = pl.kernel(
        _k1_body, out_type=(_f32((NPAD,)),) * 3, mesh=mesh, compiler_params=cp,
        scratch_types=[pltpu.VMEM((NSL,), jnp.float32)] * 7,
    )
    base_pot, local, potential = k1(bed_p, ob_p, melt_p, smelt_p)

    k2 = pl.kernel(
        _k2_body, out_type=_f32((NPAD,)), mesh=mesh, compiler_params=cp,
        scratch_types=[
            pltpu.VMEM((NPAD,), jnp.float32),
            pltpu.VMEM((NSL,), jnp.int32),
            pltpu.VMEM((NSL,), jnp.int32),
            pltpu.VMEM((NSL,), jnp.float32),
            pltpu.VMEM((NSL,), jnp.float32),
        ],
    )
    inflow = k2(base_pot, adjt_p, stat_p)

    k3 = pl.kernel(
        _k3_body, out_type=(_f32((NC * NPAD,)),) * 3, mesh=mesh, compiler_params=cp,
        scratch_types=[
            pltpu.VMEM((C,), jnp.int32),
            pltpu.VMEM((C,), jnp.int32),
            pltpu.VMEM((C,), jnp.float32),
            pltpu.VMEM((C,), jnp.int32),
            pltpu.VMEM((C,), jnp.int32),
            pltpu.VMEM((C,), jnp.float32),
            pltpu.VMEM((C,), jnp.float32),
            pltpu.VMEM((C,), jnp.float32),
            pltpu.VMEM((C,), jnp.float32),
            pltpu.VMEM((C,), jnp.float32),
            pltpu.VMEM((1568,), jnp.float32),
            pltpu.SemaphoreType.DMA,
            pltpu.SemaphoreType.DMA,
            pltpu.VMEM_SHARED((NPAD,), jnp.float32),
            pltpu.VMEM_SHARED((NPAD,), jnp.float32),
            pltpu.VMEM_SHARED((NPAD,), jnp.float32),
            pltpu.VMEM_SHARED((NPAD,), jnp.float32),
            pltpu.VMEM_SHARED((NPAD,), jnp.float32),
        ],
    )
    dis_p, vs_p, vc_p = k3(head, tail, sliding_velocity, base_pot, local)

    k4 = pl.kernel(
        _k4_body, out_type=(_f32((NPAD,)),) * 2, mesh=mesh, compiler_params=cp,
        scratch_types=(
            [pltpu.VMEM((NSL,), jnp.float32)] * 7
            + [pltpu.VMEM((NSL,), jnp.int32)]
            + [pltpu.VMEM((NSL,), jnp.float32)] * 4
        ),
    )
    discharge_node, thickness = k4(dis_p, vs_p, vc_p, local, stat_p, ob_p, inflow)

    k5 = pl.kernel(
        _k5_body, out_type=_f32((E,)), mesh=mesh, compiler_params=cp,
        scratch_types=(
            [pltpu.VMEM((C,), jnp.int32)] * 2
            + [pltpu.VMEM((C,), jnp.float32)] * 7
            + [pltpu.VMEM((1568,), jnp.float32)]
            + [pltpu.SemaphoreType.DMA]
            + [pltpu.VMEM_SHARED((NPAD,), jnp.float32)] * 3
        ),
    )
    residual = k5(head, tail, potential, thickness, discharge_node)
    return residual


# speedup vs baseline: 313.7161x; 1.0105x over previous
import functools

import jax
import jax.numpy as jnp
from jax import lax
from jax.experimental import pallas as pl
from jax.experimental.pallas import tpu as pltpu
from jax.experimental.pallas import tpu_sc as plsc

RHO_W = 1000.0
RHO_I = 917.0
G = 9.81
SEC_PER_A = 31556926.0
DX = 100.0
CELL_AREA = DX * DX
SHEET_K = 0.01
STEP_H = 0.1
SPACING = 2.0
CLOSURE = 5e-25
NEXP = 3

N = 100000
E = 1600000
K_ADJ = 8

NC = 2
NS = 16
NW = NC * NS
NPAD = 100352
NSL = NPAD // NW
EW = E // NW
NSL2 = NPAD // NS
C = 2000
NCH = EW // C
L = 16


def _mesh():
    return plsc.VectorSubcoreMesh(core_axis_name="c", subcore_axis_name="s",
                                  num_cores=NC, num_subcores=NS)


def _wid():
    return lax.axis_index("s") * NC + lax.axis_index("c")


def _rsqrt(x):
    i = plsc.bitcast(x, jnp.int32)
    i = 0x5F3759DF - lax.shift_right_logical(i, 1)
    y = plsc.bitcast(i, jnp.float32)
    for _ in range(3):
        y = y * (1.5 - 0.5 * x * y * y)
    return y


def _k1_body(bed, ob, melt, smelt, bp_out, loc_out, pot_out,
             bedv, obv, meltv, smeltv, bpv, locv, potv):
    w = _wid()
    off = w * NSL
    pltpu.sync_copy(bed.at[pl.ds(off, NSL)], bedv)
    pltpu.sync_copy(ob.at[pl.ds(off, NSL)], obv)
    pltpu.sync_copy(melt.at[pl.ds(off, NSL)], meltv)
    pltpu.sync_copy(smelt.at[pl.ds(off, NSL)], smeltv)

    def body(i, _):
        s = pl.ds(i * L, L)
        b = bedv[s]
        o = obv[s]
        bp = RHO_W * G * b + o
        bpv[s] = bp
        potv[s] = bp - o
        locv[s] = (meltv[s] * (RHO_W / RHO_I / SEC_PER_A) + smeltv[s]) * CELL_AREA
        return 0

    lax.fori_loop(0, NSL // L, body, 0)
    pltpu.sync_copy(bpv, bp_out.at[pl.ds(off, NSL)])
    pltpu.sync_copy(locv, loc_out.at[pl.ds(off, NSL)])
    pltpu.sync_copy(potv, pot_out.at[pl.ds(off, NSL)])


def _k2_body(bp, adjt, status, if_out, bptab, adjv, statv, accv, outv):
    w = _wid()
    off = w * NSL
    pltpu.sync_copy(bp, bptab)
    pltpu.sync_copy(status.at[pl.ds(off, NSL)], statv)

    def zero(i, _):
        accv[pl.ds(i * L, L)] = jnp.zeros((L,), jnp.float32)
        return 0

    lax.fori_loop(0, NSL // L, zero, 0)

    def per_j(j, _):
        joff = pl.multiple_of(j * NPAD + off, 8)
        pltpu.sync_copy(adjt.at[pl.ds(joff, NSL)], adjv)

        def per_i(i, _):
            s = pl.ds(i * L, L)
            idx = adjv[s]
            accv[s] = accv[s] + plsc.load_gather(bptab, [idx])
            return 0

        lax.fori_loop(0, NSL // L, per_i, 0)
        return 0

    lax.fori_loop(0, K_ADJ, per_j, 0)

    def fin(i, _):
        s = pl.ds(i * L, L)
        adj_pot = accv[s] * (1.0 / K_ADJ)
        mybp = bptab[pl.ds(off + i * L, L)]
        sign = jnp.where(mybp > adj_pot, 1.0, -1.0)
        outv[s] = jnp.where(statv[s] > 0, sign, 0.0)
        return 0

    lax.fori_loop(0, NSL // L, fin, 0)
    pltpu.sync_copy(outv, if_out.at[pl.ds(off, NSL)])


def _k3_body(head, tail, slide, bp, loc,
             dis_out, vs_out, vc_out,
             headv, tailv, slidev, recvv, sendv, lsendv, onesv, zv,
             bphv, bptv, bouncev, sem, ssem, bp_sh, loc_sh, dis_sh, vs_sh, vc_sh):
    c = lax.axis_index("c")
    s = lax.axis_index("s")
    w = s * NC + c
    noff = s * NSL

    def zfill(i, _):
        zv[pl.ds(i * L, L)] = jnp.zeros((L,), jnp.float32)
        onesv[pl.ds(i * L, L)] = jnp.full((L,), 1.0, jnp.float32)
        return 0

    lax.fori_loop(0, C // L, zfill, 0)

    def stripe2(k, _):
        soff = noff + k * 1568
        pltpu.sync_copy(zv.at[pl.ds(0, 1568)], dis_sh.at[pl.ds(soff, 1568)])
        pltpu.sync_copy(zv.at[pl.ds(0, 1568)], vs_sh.at[pl.ds(soff, 1568)])
        pltpu.sync_copy(zv.at[pl.ds(0, 1568)], vc_sh.at[pl.ds(soff, 1568)])
        pltpu.sync_copy(loc.at[pl.ds(soff, 1568)], bouncev)
        pltpu.sync_copy(bouncev, loc_sh.at[pl.ds(soff, 1568)])
        pltpu.sync_copy(bp.at[pl.ds(soff, 1568)], bouncev)
        pltpu.sync_copy(bouncev, bp_sh.at[pl.ds(soff, 1568)])
        return 0

    lax.fori_loop(0, 2, stripe2, 0)
    plsc.subcore_barrier()

    def chunk(ci, _):
        off = w * EW + ci * C
        lds = [
            pltpu.async_copy(head.at[pl.ds(off, C)], headv, sem),
            pltpu.async_copy(tail.at[pl.ds(off, C)], tailv, sem),
            pltpu.async_copy(slide.at[pl.ds(off, C)], slidev, sem),
        ]
        for ld_ in lds:
            ld_.wait()
        gs = [
            pltpu.async_copy(bp_sh.at[headv], bphv, sem),
            pltpu.async_copy(bp_sh.at[tailv], bptv, sem),
        ]
        for g_ in gs:
            g_.wait()

        def vb(i, _):
            sl = pl.ds(i * L, L)
            h = headv[sl]
            t = tailv[sl]
            down = bptv[sl] > bphv[sl]
            recvv[sl] = jnp.where(down, h, t)
            sendv[sl] = jnp.where(down, t, h)
            return 0

        lax.fori_loop(0, C // L, vb, 0)

        pltpu.sync_copy(loc_sh.at[sendv], lsendv)
        scs = [
            pltpu.async_copy(lsendv, dis_sh.at[recvv], ssem, add=True),
            pltpu.async_copy(slidev, vs_sh.at[headv], ssem, add=True),
            pltpu.async_copy(slidev, vs_sh.at[tailv], ssem, add=True),
            pltpu.async_copy(onesv, vc_sh.at[headv], ssem, add=True),
            pltpu.async_copy(onesv, vc_sh.at[tailv], ssem, add=True),
        ]
        for sc_ in scs:
            sc_.wait()
        return 0

    lax.fori_loop(0, NCH, chunk, 0)
    plsc.subcore_barrier()

    def out2(k, _):
        soff = noff + k * 1568
        hoff = c * NPAD + soff
        pltpu.sync_copy(dis_sh.at[pl.ds(soff, 1568)], bouncev)
        pltpu.sync_copy(bouncev, dis_out.at[pl.ds(hoff, 1568)])
        pltpu.sync_copy(vs_sh.at[pl.ds(soff, 1568)], bouncev)
        pltpu.sync_copy(bouncev, vs_out.at[pl.ds(hoff, 1568)])
        pltpu.sync_copy(vc_sh.at[pl.ds(soff, 1568)], bouncev)
        pltpu.sync_copy(bouncev, vc_out.at[pl.ds(hoff, 1568)])
        return 0

    lax.fori_loop(0, 2, out2, 0)


def _k5_body(head, tail, pot, dis_p, vs_p, vc_p, loc, status, ob, inf, res_out,
             headv, tailv, phv, ptv, thv, ttv, dhv, dtv, outv,
             bouncev, d0v, d1v, v0v, v1v, c0v, c1v, locv, statv, obv, infv,
             dnv, thnv, sem, pot_sh, th_sh, dn_sh):
    c = lax.axis_index("c")
    s = lax.axis_index("s")
    w = s * NC + c
    noff = s * NSL2

    pltpu.sync_copy(dis_p.at[pl.ds(noff, NSL2)], d0v)
    pltpu.sync_copy(dis_p.at[pl.ds(NPAD + noff, NSL2)], d1v)
    pltpu.sync_copy(vs_p.at[pl.ds(noff, NSL2)], v0v)
    pltpu.sync_copy(vs_p.at[pl.ds(NPAD + noff, NSL2)], v1v)
    pltpu.sync_copy(vc_p.at[pl.ds(noff, NSL2)], c0v)
    pltpu.sync_copy(vc_p.at[pl.ds(NPAD + noff, NSL2)], c1v)
    pltpu.sync_copy(loc.at[pl.ds(noff, NSL2)], locv)
    pltpu.sync_copy(status.at[pl.ds(noff, NSL2)], statv)
    pltpu.sync_copy(ob.at[pl.ds(noff, NSL2)], obv)
    pltpu.sync_copy(inf.at[pl.ds(noff, NSL2)], infv)

    def nbody(i, _):
        sl = pl.ds(i * L, L)
        dn = locv[sl] + d0v[sl] + d1v[sl]
        dn = jnp.where(statv[sl] != 0, 0.0, dn)
        dbits = plsc.bitcast(dn, jnp.int32)
        dbits = jnp.where(infv[sl] > 0.5, dbits | jnp.int32(-2147483648), dbits)
        dnv[sl] = plsc.bitcast(dbits, jnp.float32)
        vsum = v0v[sl] + v1v[sl]
        vcnt = c0v[sl] + c1v[sl]
        sn = jnp.abs(vsum / jnp.maximum(vcnt, 1.0)) * (1.0 / SEC_PER_A)
        p = obv[sl]
        thnv[sl] = sn * sn * STEP_H / (CLOSURE * p * p * p * (SPACING * SPACING) + 1e-30)
        return 0

    lax.fori_loop(0, NSL2 // L, nbody, 0)
    pltpu.sync_copy(dnv, dn_sh.at[pl.ds(noff, NSL2)])
    pltpu.sync_copy(thnv, th_sh.at[pl.ds(noff, NSL2)])
    pltpu.sync_copy(pot.at[pl.ds(noff, NSL2)], bouncev)
    pltpu.sync_copy(bouncev, pot_sh.at[pl.ds(noff, NSL2)])
    plsc.subcore_barrier()

    def chunk(ci, _):
        off = w * EW + ci * C
        lds = [
            pltpu.async_copy(head.at[pl.ds(off, C)], headv, sem),
            pltpu.async_copy(tail.at[pl.ds(off, C)], tailv, sem),
        ]
        for ld_ in lds:
            ld_.wait()
        cps = [
            pltpu.async_copy(pot_sh.at[headv], phv, sem),
            pltpu.async_copy(pot_sh.at[tailv], ptv, sem),
            pltpu.async_copy(th_sh.at[headv], thv, sem),
            pltpu.async_copy(th_sh.at[tailv], ttv, sem),
            pltpu.async_copy(dn_sh.at[headv], dhv, sem),
            pltpu.async_copy(dn_sh.at[tailv], dtv, sem),
        ]
        for cp_ in cps:
            cp_.wait()

        def vb(i, _):
            sl = pl.ds(i * L, L)
            hl = 0.5 * (thv[sl] + ttv[sl])
            g = (phv[sl] - ptv[sl]) * (1.0 / DX)
            a = jnp.abs(g) + 1e-12
            r = _rsqrt(a)
            q = _rsqrt(_rsqrt(hl))
            flux = (-SHEET_K) * hl * q * r * g
            dhb = plsc.bitcast(dhv[sl], jnp.int32)
            dtb = plsc.bitcast(dtv[sl], jnp.int32)
            gate = (dhb < 0) | (dtb < 0)
            flux = jnp.where(gate, 0.0, flux)
            d = 0.5 * (jnp.abs(dhv[sl]) + jnp.abs(dtv[sl]))
            outv[sl] = jnp.abs(flux - d)
            return 0

        lax.fori_loop(0, C // L, vb, 0)
        pltpu.sync_copy(outv, res_out.at[pl.ds(off, C)])
        return 0

    lax.fori_loop(0, NCH, chunk, 0)


def _f32(shape):
    return jax.ShapeDtypeStruct(shape, jnp.float32)


def kernel(edge_index, adjacent_nodes, status_at_node, bedrock_elevation,
           overburden_pressure, melt_rate, surface_melt_rate, sliding_velocity):
    head = edge_index[0]
    tail = edge_index[1]
    pad = NPAD - N
    bed_p = jnp.pad(bedrock_elevation, (0, pad))
    ob_p = jnp.pad(overburden_pressure, (0, pad))
    melt_p = jnp.pad(melt_rate, (0, pad))
    smelt_p = jnp.pad(surface_melt_rate, (0, pad))
    stat_p = jnp.pad(status_at_node, (0, pad))
    adjt_p = jnp.pad(adjacent_nodes.T, ((0, 0), (0, pad))).reshape(-1)

    mesh = _mesh()
    cp = pltpu.CompilerParams(needs_layout_passes=False)

    k1 = pl.kernel(
        _k1_body, out_type=(_f32((NPAD,)),) * 3, mesh=mesh, compiler_params=cp,
        scratch_types=[pltpu.VMEM((NSL,), jnp.float32)] * 7,
    )
    base_pot, local, potential = k1(bed_p, ob_p, melt_p, smelt_p)

    k2 = pl.kernel(
        _k2_body, out_type=_f32((NPAD,)), mesh=mesh, compiler_params=cp,
        scratch_types=[
            pltpu.VMEM((NPAD,), jnp.float32),
            pltpu.VMEM((NSL,), jnp.int32),
            pltpu.VMEM((NSL,), jnp.int32),
            pltpu.VMEM((NSL,), jnp.float32),
            pltpu.VMEM((NSL,), jnp.float32),
        ],
    )
    inflow = k2(base_pot, adjt_p, stat_p)

    k3 = pl.kernel(
        _k3_body, out_type=(_f32((NC * NPAD,)),) * 3, mesh=mesh, compiler_params=cp,
        scratch_types=[
            pltpu.VMEM((C,), jnp.int32),
            pltpu.VMEM((C,), jnp.int32),
            pltpu.VMEM((C,), jnp.float32),
            pltpu.VMEM((C,), jnp.int32),
            pltpu.VMEM((C,), jnp.int32),
            pltpu.VMEM((C,), jnp.float32),
            pltpu.VMEM((C,), jnp.float32),
            pltpu.VMEM((C,), jnp.float32),
            pltpu.VMEM((C,), jnp.float32),
            pltpu.VMEM((C,), jnp.float32),
            pltpu.VMEM((1568,), jnp.float32),
            pltpu.SemaphoreType.DMA,
            pltpu.SemaphoreType.DMA,
            pltpu.VMEM_SHARED((NPAD,), jnp.float32),
            pltpu.VMEM_SHARED((NPAD,), jnp.float32),
            pltpu.VMEM_SHARED((NPAD,), jnp.float32),
            pltpu.VMEM_SHARED((NPAD,), jnp.float32),
            pltpu.VMEM_SHARED((NPAD,), jnp.float32),
        ],
    )
    dis_p, vs_p, vc_p = k3(head, tail, sliding_velocity, base_pot, local)

    k5 = pl.kernel(
        _k5_body, out_type=_f32((E,)), mesh=mesh, compiler_params=cp,
        scratch_types=(
            [pltpu.VMEM((C,), jnp.int32)] * 2
            + [pltpu.VMEM((C,), jnp.float32)] * 7
            + [pltpu.VMEM((NSL2,), jnp.float32)] * 8
            + [pltpu.VMEM((NSL2,), jnp.int32)]
            + [pltpu.VMEM((NSL2,), jnp.float32)] * 4
            + [pltpu.SemaphoreType.DMA]
            + [pltpu.VMEM_SHARED((NPAD,), jnp.float32)] * 3
        ),
    )
    residual = k5(head, tail, potential, dis_p, vs_p, vc_p, local,
                  stat_p, ob_p, inflow)
    return residual
